# baseline (speedup 1.0000x reference)
import jax
import jax.numpy as jnp
import numpy as np
from jax.experimental import pallas as pl
from jax.experimental.pallas import tpu as pltpu

_H = _W = 128
_RING_BOUNDS = [(0, 16), (16, 32), (32, 48), (48, 64)]
_NRINGS = 4
_BB = 2
_ROWS = _BB * 3
_YS = 8


def _ring_masks():
    yy, xx = np.mgrid[0:_H, 0:_W]
    d2 = (xx - 64) ** 2 + (yy - 64) ** 2
    ms = []
    for r0, r1 in _RING_BOUNDS:
        ms.append(((d2 <= r1 * r1) & (d2 > r0 * r0)).astype(np.float32))
    return np.stack(ms)


_MASKS_NP = _ring_masks()
_RING_N = [int(m.sum()) for m in _MASKS_NP]
_RANK_A = [n // 2 for n in _RING_N]


def _body(x_ref, m_ref, out_ref, xs_ref):
    nchunks = _H // _YS

    s1acc = [jnp.zeros((_ROWS, _YS, _W), jnp.float32) for _ in range(_NRINGS)]
    s2acc = [jnp.zeros((_ROWS, _YS, _W), jnp.float32) for _ in range(_NRINGS)]
    for c in range(nchunks):
        ys = slice(c * _YS, (c + 1) * _YS)
        x = jnp.floor(x_ref[:, :, ys, :].reshape(_ROWS, _YS, _W) * 255.0)
        xs_ref[:, ys, :] = x
        xsq = x * x
        for r in range(_NRINGS):
            m = m_ref[r, ys, :][None]
            s1acc[r] += x * m
            s2acc[r] += xsq * m

    def _tot(acc):
        return jnp.sum(acc.sum(axis=1), axis=1, keepdims=True)

    s1 = [_tot(a) for a in s1acc]
    s2 = [_tot(a) for a in s2acc]

    def bisect_step(_, carry):
        los, his = carry
        mids = [jnp.floor((los[r] + his[r]) * 0.5) for r in range(_NRINGS)]
        midb = [mids[r][:, :, None] for r in range(_NRINGS)]
        accs = [jnp.zeros((_ROWS, _YS, _W), jnp.float32) for _ in range(_NRINGS)]
        for c in range(nchunks):
            ys = slice(c * _YS, (c + 1) * _YS)
            x = xs_ref[:, ys, :]
            for r in range(_NRINGS):
                m = m_ref[r, ys, :][None]
                accs[r] += jnp.where(x <= midb[r], m, 0.0)
        new_los, new_his = [], []
        for r in range(_NRINGS):
            cnt = _tot(accs[r])
            ge = cnt >= float(_RANK_A[r])
            new_his.append(jnp.where(ge, mids[r], his[r]))
            new_los.append(jnp.where(ge, los[r], mids[r] + 1.0))
        return new_los, new_his

    los = [jnp.zeros((_ROWS, 1), jnp.float32) for _ in range(_NRINGS)]
    his = [jnp.full((_ROWS, 1), 255.0, jnp.float32) for _ in range(_NRINGS)]
    los, his = jax.lax.fori_loop(0, 8, bisect_step, (los, his))
    mas = los

    cacc = [jnp.zeros((_ROWS, _YS, _W), jnp.float32) for _ in range(_NRINGS)]
    macc = [jnp.full((_ROWS, _YS, _W), 1e9, jnp.float32) for _ in range(_NRINGS)]
    masb = [mas[r][:, :, None] for r in range(_NRINGS)]
    for c in range(nchunks):
        ys = slice(c * _YS, (c + 1) * _YS)
        x = xs_ref[:, ys, :]
        for r in range(_NRINGS):
            m = m_ref[r, ys, :][None]
            le = x <= masb[r]
            cacc[r] += jnp.where(le, m, 0.0)
            macc[r] = jnp.minimum(
                macc[r], jnp.where((~le) & (m > 0.5), x, 1e9)
            )
    for r in range(_NRINGS):
        cnt_a = _tot(cacc[r])
        min_above = jnp.min(
            jnp.min(macc[r], axis=1), axis=1, keepdims=True
        )
        mb = jnp.where(cnt_a >= float(_RANK_A[r] + 1), mas[r], min_above)
        med = 0.5 * (mas[r] + mb)
        inv_n = 1.0 / float(_RING_N[r])
        mean = s1[r] * inv_n
        var = s2[r] * inv_n - mean * mean
        std = jnp.sqrt(jnp.maximum(var, 0.0))
        out_ref[0, :, 3 * r : 3 * r + 1] = mean
        out_ref[0, :, 3 * r + 1 : 3 * r + 2] = std
        out_ref[0, :, 3 * r + 2 : 3 * r + 3] = med


@jax.jit
def kernel(image_tensor):
    b = image_tensor.shape[0]
    rows = b * 3
    masks = jnp.asarray(_MASKS_NP)
    grid = (b // _BB,)
    out = pl.pallas_call(
        _body,
        grid=grid,
        in_specs=[
            pl.BlockSpec((_BB, 3, _H, _W), lambda i: (i, 0, 0, 0)),
            pl.BlockSpec((_NRINGS, _H, _W), lambda i: (0, 0, 0)),
        ],
        out_specs=pl.BlockSpec((1, _ROWS, 12), lambda i: (i, 0, 0)),
        out_shape=jax.ShapeDtypeStruct((b // _BB, _ROWS, 12), jnp.float32),
        scratch_shapes=[pltpu.VMEM((_ROWS, _H, _W), jnp.float32)],
        compiler_params=pltpu.CompilerParams(
            dimension_semantics=("arbitrary",),
        ),
    )(image_tensor, masks)
    return (
        out.reshape(b, 3, _NRINGS, 3)
        .transpose(0, 2, 3, 1)
        .reshape(b, _NRINGS, 9)
    )


# speedup vs baseline: 1.0899x; 1.0899x over previous
import jax
import jax.numpy as jnp
import numpy as np
from jax.experimental import pallas as pl
from jax.experimental.pallas import tpu as pltpu

_H = _W = 128
_RING_BOUNDS = [(0, 16), (16, 32), (32, 48), (48, 64)]
_NRINGS = 4
_BB = 4
_ROWS = _BB * 3
_YS = 16


def _ring_masks():
    yy, xx = np.mgrid[0:_H, 0:_W]
    d2 = (xx - 64) ** 2 + (yy - 64) ** 2
    ms = []
    for r0, r1 in _RING_BOUNDS:
        ms.append(((d2 <= r1 * r1) & (d2 > r0 * r0)).astype(np.float32))
    return np.stack(ms)


_MASKS_NP = _ring_masks()
_RING_N = [int(m.sum()) for m in _MASKS_NP]
_RANK_A = [n // 2 for n in _RING_N]


def _body(x_ref, m_ref, mbf_ref, out_ref, xs_ref):
    nchunks = _H // _YS

    s1acc = [jnp.zeros((_ROWS, _YS, _W), jnp.float32) for _ in range(_NRINGS)]
    s2acc = [jnp.zeros((_ROWS, _YS, _W), jnp.float32) for _ in range(_NRINGS)]
    for c in range(nchunks):
        ys = slice(c * _YS, (c + 1) * _YS)
        x = jnp.floor(x_ref[:, :, ys, :].reshape(_ROWS, _YS, _W) * 255.0)
        xs_ref[:, ys, :] = x.astype(jnp.bfloat16)
        for r in range(_NRINGS):
            m = m_ref[r, ys, :][None]
            xm = x * m
            s1acc[r] += xm
            s2acc[r] += xm * x

    def _tot(acc):
        return jnp.sum(acc.sum(axis=1), axis=1, keepdims=True)

    s1 = [_tot(a) for a in s1acc]
    s2 = [_tot(a) for a in s2acc]

    zero_cnt = [jnp.zeros((_ROWS, 1), jnp.float32) for _ in range(_NRINGS)]

    def bisect_step(_, carry):
        los, his, _cnts = carry
        mids = [jnp.floor((los[r] + his[r]) * 0.5) for r in range(_NRINGS)]
        midb = [mids[r].astype(jnp.bfloat16)[:, :, None] for r in range(_NRINGS)]
        accs = [jnp.zeros((_ROWS, _YS, _W), jnp.bfloat16) for _ in range(_NRINGS)]
        for c in range(nchunks):
            ys = slice(c * _YS, (c + 1) * _YS)
            x = xs_ref[:, ys, :]
            for r in range(_NRINGS):
                m = mbf_ref[r, ys, :][None]
                accs[r] += jnp.where(x <= midb[r], m, jnp.bfloat16(0))
        new_los, new_his, cnts = [], [], []
        for r in range(_NRINGS):
            cnt = _tot(accs[r].astype(jnp.float32))
            cnts.append(cnt)
            ge = cnt >= float(_RANK_A[r])
            new_his.append(jnp.where(ge, mids[r], his[r]))
            new_los.append(jnp.where(ge, los[r], mids[r] + 1.0))
        return new_los, new_his, cnts

    los = [jnp.zeros((_ROWS, 1), jnp.float32) for _ in range(_NRINGS)]
    his = [jnp.full((_ROWS, 1), 255.0, jnp.float32) for _ in range(_NRINGS)]
    los, his, cnts = jax.lax.fori_loop(0, 9, bisect_step, (los, his, zero_cnt))
    mas = los

    thrb = [(255.0 - mas[r]).astype(jnp.bfloat16)[:, :, None] for r in range(_NRINGS)]
    maxy = [jnp.zeros((_ROWS, _YS, _W), jnp.bfloat16) for _ in range(_NRINGS)]
    for c in range(nchunks):
        ys = slice(c * _YS, (c + 1) * _YS)
        y = jnp.bfloat16(255) - xs_ref[:, ys, :]
        for r in range(_NRINGS):
            ym = y * mbf_ref[r, ys, :][None]
            maxy[r] = jnp.maximum(
                maxy[r], jnp.where(ym < thrb[r], ym, jnp.bfloat16(0))
            )
    for r in range(_NRINGS):
        my = jnp.max(jnp.max(maxy[r].astype(jnp.float32), axis=1),
                     axis=1, keepdims=True)
        min_above = 255.0 - my
        mb = jnp.where(cnts[r] >= float(_RANK_A[r] + 1), mas[r], min_above)
        med = 0.5 * (mas[r] + mb)
        inv_n = 1.0 / float(_RING_N[r])
        mean = s1[r] * inv_n
        var = s2[r] * inv_n - mean * mean
        std = jnp.sqrt(jnp.maximum(var, 0.0))
        out_ref[0, :, 3 * r : 3 * r + 1] = mean
        out_ref[0, :, 3 * r + 1 : 3 * r + 2] = std
        out_ref[0, :, 3 * r + 2 : 3 * r + 3] = med


@jax.jit
def kernel(image_tensor):
    b = image_tensor.shape[0]
    masks = jnp.asarray(_MASKS_NP)
    masks_bf = jnp.asarray(_MASKS_NP.astype(np.float32)).astype(jnp.bfloat16)
    grid = (b // _BB,)
    out = pl.pallas_call(
        _body,
        grid=grid,
        in_specs=[
            pl.BlockSpec((_BB, 3, _H, _W), lambda i: (i, 0, 0, 0)),
            pl.BlockSpec((_NRINGS, _H, _W), lambda i: (0, 0, 0)),
            pl.BlockSpec((_NRINGS, _H, _W), lambda i: (0, 0, 0)),
        ],
        out_specs=pl.BlockSpec((1, _ROWS, 12), lambda i: (i, 0, 0)),
        out_shape=jax.ShapeDtypeStruct((b // _BB, _ROWS, 12), jnp.float32),
        scratch_shapes=[pltpu.VMEM((_ROWS, _H, _W), jnp.bfloat16)],
        compiler_params=pltpu.CompilerParams(
            dimension_semantics=("arbitrary",),
        ),
    )(image_tensor, masks, masks_bf)
    return (
        out.reshape(b, 3, _NRINGS, 3)
        .transpose(0, 2, 3, 1)
        .reshape(b, _NRINGS, 9)
    )


# speedup vs baseline: 1.3338x; 1.2238x over previous
import jax
import jax.numpy as jnp
import numpy as np
from jax.experimental import pallas as pl
from jax.experimental.pallas import tpu as pltpu

_H = _W = 128
_RING_BOUNDS = [(0, 16), (16, 32), (32, 48), (48, 64)]
_NRINGS = 4
_BB = 4
_ROWS = _BB * 3
_YS = 16

_CLO = [(64 - r1) // _YS for _, r1 in _RING_BOUNDS]
_CHI = [min(64 + r1, _H - 1) // _YS + 1 for _, r1 in _RING_BOUNDS]
_CLO8 = [(64 - r1) // 8 for _, r1 in _RING_BOUNDS]
_CHI8 = [min(64 + r1, _H - 1) // 8 + 1 for _, r1 in _RING_BOUNDS]


def _ring_masks():
    yy, xx = np.mgrid[0:_H, 0:_W]
    d2 = (xx - 64) ** 2 + (yy - 64) ** 2
    ms = []
    for r0, r1 in _RING_BOUNDS:
        ms.append(((d2 <= r1 * r1) & (d2 > r0 * r0)).astype(np.float32))
    return np.stack(ms)


_MASKS_NP = _ring_masks()
_RING_N = [int(m.sum()) for m in _MASKS_NP]
_RANK_A = [n // 2 for n in _RING_N]


def _body(x_ref, m_ref, mbf_ref, out_ref, xb_ref, xf_ref, x2_ref):
    nchunks = _H // _YS

    for c in range(nchunks):
        ys = slice(c * _YS, (c + 1) * _YS)
        x = jnp.floor(x_ref[:, :, ys, :].reshape(_ROWS, _YS, _W) * 255.0)
        xf_ref[:, ys, :] = x
        x2_ref[:, ys, :] = x * x
        xb_ref[:, ys, :] = x.astype(jnp.bfloat16)

    s1, s2 = [], []
    for r in range(_NRINGS):
        s1a = jnp.zeros((_ROWS, 8, _W), jnp.float32)
        s2a = jnp.zeros((_ROWS, 8, _W), jnp.float32)
        for c in range(_CLO8[r], _CHI8[r]):
            ys = slice(c * 8, (c + 1) * 8)
            m = m_ref[r, ys, :][None]
            s1a += xf_ref[:, ys, :] * m
            s2a += x2_ref[:, ys, :] * m
        s1.append(jnp.sum(s1a.sum(axis=1), axis=1, keepdims=True))
        s2.append(jnp.sum(s2a.sum(axis=1), axis=1, keepdims=True))

    zero_cnt = [jnp.zeros((_ROWS, 1), jnp.float32) for _ in range(_NRINGS)]

    def bisect_step(_, carry):
        los, his, _cnts = carry
        new_los, new_his, cnts = [], [], []
        for r in range(_NRINGS):
            mid = jnp.floor((los[r] + his[r]) * 0.5)
            midb = mid.astype(jnp.bfloat16)[:, :, None]
            acc = jnp.zeros((_ROWS, _YS, _W), jnp.bfloat16)
            for c in range(_CLO[r], _CHI[r]):
                ys = slice(c * _YS, (c + 1) * _YS)
                x = xb_ref[:, ys, :]
                m = mbf_ref[r, ys, :][None]
                acc += jnp.where(x <= midb, m, jnp.bfloat16(0))
            cnt = jnp.sum(acc.astype(jnp.float32).sum(axis=1),
                          axis=1, keepdims=True)
            cnts.append(cnt)
            ge = cnt >= float(_RANK_A[r])
            new_his.append(jnp.where(ge, mid, his[r]))
            new_los.append(jnp.where(ge, los[r], mid + 1.0))
        return new_los, new_his, cnts

    los = [jnp.zeros((_ROWS, 1), jnp.float32) for _ in range(_NRINGS)]
    his = [jnp.full((_ROWS, 1), 255.0, jnp.float32) for _ in range(_NRINGS)]
    los, his, cnts = jax.lax.fori_loop(0, 9, bisect_step, (los, his, zero_cnt))
    mas = los

    for r in range(_NRINGS):
        thrb = (255.0 - mas[r]).astype(jnp.bfloat16)[:, :, None]
        maxy = jnp.zeros((_ROWS, _YS, _W), jnp.bfloat16)
        for c in range(_CLO[r], _CHI[r]):
            ys = slice(c * _YS, (c + 1) * _YS)
            ym = (jnp.bfloat16(255) - xb_ref[:, ys, :]) * mbf_ref[r, ys, :][None]
            maxy = jnp.maximum(maxy, jnp.where(ym < thrb, ym, jnp.bfloat16(0)))
        my = jnp.max(jnp.max(maxy.astype(jnp.float32), axis=1),
                     axis=1, keepdims=True)
        min_above = 255.0 - my
        mb = jnp.where(cnts[r] >= float(_RANK_A[r] + 1), mas[r], min_above)
        med = 0.5 * (mas[r] + mb)
        inv_n = 1.0 / float(_RING_N[r])
        mean = s1[r] * inv_n
        var = s2[r] * inv_n - mean * mean
        std = jnp.sqrt(jnp.maximum(var, 0.0))
        out_ref[0, :, 3 * r : 3 * r + 1] = mean
        out_ref[0, :, 3 * r + 1 : 3 * r + 2] = std
        out_ref[0, :, 3 * r + 2 : 3 * r + 3] = med


@jax.jit
def kernel(image_tensor):
    b = image_tensor.shape[0]
    masks = jnp.asarray(_MASKS_NP)
    masks_bf = jnp.asarray(_MASKS_NP).astype(jnp.bfloat16)
    grid = (b // _BB,)
    out = pl.pallas_call(
        _body,
        grid=grid,
        in_specs=[
            pl.BlockSpec((_BB, 3, _H, _W), lambda i: (i, 0, 0, 0)),
            pl.BlockSpec((_NRINGS, _H, _W), lambda i: (0, 0, 0)),
            pl.BlockSpec((_NRINGS, _H, _W), lambda i: (0, 0, 0)),
        ],
        out_specs=pl.BlockSpec((1, _ROWS, 12), lambda i: (i, 0, 0)),
        out_shape=jax.ShapeDtypeStruct((b // _BB, _ROWS, 12), jnp.float32),
        scratch_shapes=[
            pltpu.VMEM((_ROWS, _H, _W), jnp.bfloat16),
            pltpu.VMEM((_ROWS, _H, _W), jnp.float32),
            pltpu.VMEM((_ROWS, _H, _W), jnp.float32),
        ],
        compiler_params=pltpu.CompilerParams(
            dimension_semantics=("arbitrary",),
        ),
    )(image_tensor, masks, masks_bf)
    return (
        out.reshape(b, 3, _NRINGS, 3)
        .transpose(0, 2, 3, 1)
        .reshape(b, _NRINGS, 9)
    )
